# baseline (speedup 1.0000x reference)
import functools

import numpy as np
import jax
import jax.numpy as jnp
from jax import lax
from jax.experimental import pallas as pl
from jax.experimental.pallas import tpu as pltpu
from jax.experimental.pallas import tpu_sc as plsc

N = 10000
D = 128
NC = 2
NS = 16
NW = NC * NS
CH = 128
NBUF = 2
N_PAD = 10112
ROWS_PER_TILE = N_PAD // NS
KSTG = 24
FRAC0 = 0.5
BULK_ALIGN = 256
TAIL_ALIGN = 256


def _core_split(k_per_pair, align=8):
    a = int(round(FRAC0 * k_per_pair / align)) * align
    a = max(0, min(a, k_per_pair))
    return a, k_per_pair - a


def _matmul_body(x_ref, wu_ref, wd_ref, wb_ref, yu_ref, yd_ref, yb_ref):
    xb = x_ref[...]
    yu_ref[...] = jnp.dot(xb, wu_ref[...], preferred_element_type=jnp.float32)
    yd_ref[...] = jnp.dot(xb, wd_ref[...], preferred_element_type=jnp.float32)
    yb_ref[...] = jnp.dot(xb, wb_ref[...], preferred_element_type=jnp.float32)


def _transform(x, W_up, W_down, W_b):
    blk = 1000
    w_spec = pl.BlockSpec((D, D), lambda i: (0, 0))
    row_spec = pl.BlockSpec((blk, D), lambda i: (i, 0))
    return pl.pallas_call(
        _matmul_body,
        grid=(N // blk,),
        in_specs=[row_spec, w_spec, w_spec, w_spec],
        out_specs=[row_spec, row_spec, row_spec],
        out_shape=[jax.ShapeDtypeStruct((N, D), jnp.float32)] * 3,
    )(x, W_up, W_down, W_b)


def _sc_scatter_body(yu, yd, yb, eu, edn, eb, tu, tdn, tb, zeros,
                     p0, p1, acc, idx_s, idx_d, bufs, gsems, ssems):
    c = lax.axis_index("c")
    s = lax.axis_index("s")

    pltpu.sync_copy(zeros, acc.at[pl.ds(s * ROWS_PER_TILE, ROWS_PER_TILE)])
    plsc.subcore_barrier()

    def run_stage(y, e_hbm, base, k_rows):
        pltpu.sync_copy(e_hbm.at[0, pl.ds(base, k_rows)], idx_s.at[pl.ds(0, k_rows)])
        pltpu.sync_copy(e_hbm.at[1, pl.ds(base, k_rows)], idx_d.at[pl.ds(0, k_rows)])

        def g_start(j, b):
            pltpu.async_copy(y.at[idx_s.at[j]], bufs[b], gsems[b])

        def g_wait(b):
            pltpu.make_async_copy(y.at[idx_s.at[0]], bufs[b], gsems[b]).wait()

        def s_start(j, b):
            pltpu.make_async_copy(bufs[b], acc.at[idx_d.at[j]], ssems[b]).start(add=True)

        def s_wait(b):
            pltpu.make_async_copy(bufs[b], acc.at[idx_d.at[0]], ssems[b]).wait()

        g_start(0, 0)

        def body(i, _):
            for u in range(NBUF):
                j = NBUF * i + u
                b = u
                g_wait(b)
                s_start(j, b)
                bn = 1 - b

                @pl.when(j + 1 < k_rows)
                def _():
                    @pl.when(j >= 1)
                    def _():
                        s_wait(bn)

                    g_start(j + 1, bn)

            return _

        lax.fori_loop(0, k_rows // NBUF, body, None)
        for b in range(NBUF):
            s_wait(b)

    def run_span(y, e_hbm, a_rows, base):
        for st in range(0, a_rows, KSTG):
            k = min(KSTG, a_rows - st)
            run_stage(y, e_hbm, base + st, k)

    work = ((yu, eu, tu), (yd, edn, tdn), (yb, eb, tb))

    @pl.when(c == 0)
    def _():
        for y, e, t in work:
            a, _b = _core_split(e.shape[1] // NS)
            if a:
                run_span(y, e, a, s * a)
            at, _bt = _core_split(t.shape[1] // NS)
            if at:
                run_span(y, t, at, s * at)

    @pl.when(c == 1)
    def _():
        for y, e, t in work:
            a, b = _core_split(e.shape[1] // NS)
            if b:
                run_span(y, e, b, NS * a + s * b)
            at, bt = _core_split(t.shape[1] // NS)
            if bt:
                run_span(y, t, bt, NS * at + s * bt)

    plsc.subcore_barrier()
    rows = pl.ds(s * ROWS_PER_TILE, ROWS_PER_TILE)

    @pl.when(c == 0)
    def _():
        pltpu.sync_copy(acc.at[rows], p0.at[rows])

    @pl.when(c == 1)
    def _():
        pltpu.sync_copy(acc.at[rows], p1.at[rows])


def _sc_scatter(yu, yd, yb, eu, edn, eb, tu, tdn, tb, zeros):
    mesh = plsc.VectorSubcoreMesh(core_axis_name="c", subcore_axis_name="s")
    f = pl.kernel(
        _sc_scatter_body,
        out_type=(jax.ShapeDtypeStruct((N_PAD, D), jnp.float32),
                  jax.ShapeDtypeStruct((N_PAD, D), jnp.float32)),
        mesh=mesh,
        scratch_types=[
            pltpu.VMEM_SHARED((N_PAD, D), jnp.float32),
            pltpu.VMEM((KSTG, CH), jnp.int32),
            pltpu.VMEM((KSTG, CH), jnp.int32),
            [pltpu.VMEM((CH, D), jnp.float32)] * NBUF,
            [pltpu.SemaphoreType.DMA] * NBUF,
            [pltpu.SemaphoreType.DMA] * NBUF,
        ],
    )
    return f(yu, yd, yb, eu, edn, eb, tu, tdn, tb, zeros)


def _combine_body(p0_ref, p1_ref, b_ref, o_ref):
    o_ref[...] = p0_ref[...] + p1_ref[...] + b_ref[...]


def _combine(p0, p1, bias):
    blk = 1000
    row_spec = pl.BlockSpec((blk, D), lambda i: (i, 0))
    return pl.pallas_call(
        _combine_body,
        grid=(N // blk,),
        in_specs=[row_spec, row_spec, pl.BlockSpec((1, D), lambda i: (0, 0))],
        out_specs=pl.BlockSpec((blk, D), lambda i: (i, 0)),
        out_shape=jax.ShapeDtypeStruct((N, D), jnp.float32),
    )(p0, p1, bias)


def _split_bulk_tail(idx):
    e = idx.shape[1]
    rows = e // CH
    bulk_rows = (rows // BULK_ALIGN) * BULK_ALIGN
    eb = bulk_rows * CH
    bulk = idx[:, :eb].astype(jnp.int32).reshape(2, bulk_rows, CH)
    tail_e = e - eb
    tail_rows = -(-(tail_e // CH + (1 if tail_e % CH else 0)) // TAIL_ALIGN) * TAIL_ALIGN
    tail_rows = max(tail_rows, TAIL_ALIGN)
    n_pad = tail_rows * CH - tail_e
    pad_s = jnp.asarray(np.arange(n_pad) % (N_PAD - N), jnp.int32)
    pad_d = pad_s + N
    tail = jnp.stack([
        jnp.concatenate([idx[0, eb:].astype(jnp.int32), pad_s]),
        jnp.concatenate([idx[1, eb:].astype(jnp.int32), pad_d]),
    ]).reshape(2, tail_rows, CH)
    return bulk, tail


def kernel(x, up_index, down_index, boundary_index, W_up, W_down, W_b, bias):
    eu, tu = _split_bulk_tail(up_index)
    edn, tdn = _split_bulk_tail(down_index)
    eb, tb = _split_bulk_tail(boundary_index)
    zeros = jnp.zeros((ROWS_PER_TILE, D), jnp.float32)

    yu, yd, yb = _transform(x, W_up, W_down, W_b)
    p0, p1 = _sc_scatter(yu, yd, yb, eu, edn, eb, tu, tdn, tb, zeros)
    return _combine(p0, p1, bias.reshape(1, D))


# speedup vs baseline: 1.0237x; 1.0237x over previous
import functools

import numpy as np
import jax
import jax.numpy as jnp
from jax import lax
from jax.experimental import pallas as pl
from jax.experimental.pallas import tpu as pltpu
from jax.experimental.pallas import tpu_sc as plsc

N = 10000
D = 128
NC = 2
NS = 16
NW = NC * NS
CH = 128
NBUF = 2
N_PAD = 10112
ROWS_PER_TILE = N_PAD // NS
KSTG = 40
FRAC0 = 0.5
BULK_ALIGN = 256
TAIL_ALIGN = 256


def _core_split(k_per_pair, align=8):
    a = int(round(FRAC0 * k_per_pair / align)) * align
    a = max(0, min(a, k_per_pair))
    return a, k_per_pair - a


def _matmul_body(x_ref, wu_ref, wd_ref, wb_ref, yu_ref, yd_ref, yb_ref):
    xb = x_ref[...]
    yu_ref[...] = jnp.dot(xb, wu_ref[...], preferred_element_type=jnp.float32)
    yd_ref[...] = jnp.dot(xb, wd_ref[...], preferred_element_type=jnp.float32)
    yb_ref[...] = jnp.dot(xb, wb_ref[...], preferred_element_type=jnp.float32)


def _transform(x, W_up, W_down, W_b):
    blk = 2000
    w_spec = pl.BlockSpec((D, D), lambda i: (0, 0))
    row_spec = pl.BlockSpec((blk, D), lambda i: (i, 0))
    return pl.pallas_call(
        _matmul_body,
        grid=(N // blk,),
        in_specs=[row_spec, w_spec, w_spec, w_spec],
        out_specs=[row_spec, row_spec, row_spec],
        out_shape=[jax.ShapeDtypeStruct((N, D), jnp.float32)] * 3,
    )(x, W_up, W_down, W_b)


def _sc_scatter_body(yu, yd, yb, eu, edn, eb, tu, tdn, tb, zeros,
                     p0, p1, acc, idx_s, idx_d, bufs, gsems, ssems):
    c = lax.axis_index("c")
    s = lax.axis_index("s")

    pltpu.sync_copy(zeros, acc.at[pl.ds(s * ROWS_PER_TILE, ROWS_PER_TILE)])
    plsc.subcore_barrier()

    def run_stage(y, e_hbm, base, k_rows):
        pltpu.sync_copy(e_hbm.at[0, pl.ds(base, k_rows)], idx_s.at[pl.ds(0, k_rows)])
        pltpu.sync_copy(e_hbm.at[1, pl.ds(base, k_rows)], idx_d.at[pl.ds(0, k_rows)])

        def g_start(j, b):
            pltpu.async_copy(y.at[idx_s.at[j]], bufs[b], gsems[b])

        def g_wait(b):
            pltpu.make_async_copy(y.at[idx_s.at[0]], bufs[b], gsems[b]).wait()

        def s_start(j, b):
            pltpu.make_async_copy(bufs[b], acc.at[idx_d.at[j]], ssems[b]).start(add=True)

        def s_wait(b):
            pltpu.make_async_copy(bufs[b], acc.at[idx_d.at[0]], ssems[b]).wait()

        g_start(0, 0)

        def body(i, _):
            for u in range(NBUF):
                j = NBUF * i + u
                b = u
                g_wait(b)
                s_start(j, b)
                bn = 1 - b

                @pl.when(j + 1 < k_rows)
                def _():
                    @pl.when(j >= 1)
                    def _():
                        s_wait(bn)

                    g_start(j + 1, bn)

            return _

        lax.fori_loop(0, k_rows // NBUF, body, None)
        for b in range(NBUF):
            s_wait(b)

    def run_span(y, e_hbm, a_rows, base):
        for st in range(0, a_rows, KSTG):
            k = min(KSTG, a_rows - st)
            run_stage(y, e_hbm, base + st, k)

    work = ((yu, eu, tu), (yd, edn, tdn), (yb, eb, tb))

    @pl.when(c == 0)
    def _():
        for y, e, t in work:
            a, _b = _core_split(e.shape[1] // NS)
            if a:
                run_span(y, e, a, s * a)
            at, _bt = _core_split(t.shape[1] // NS)
            if at:
                run_span(y, t, at, s * at)

    @pl.when(c == 1)
    def _():
        for y, e, t in work:
            a, b = _core_split(e.shape[1] // NS)
            if b:
                run_span(y, e, b, NS * a + s * b)
            at, bt = _core_split(t.shape[1] // NS)
            if bt:
                run_span(y, t, bt, NS * at + s * bt)

    plsc.subcore_barrier()
    rows = pl.ds(s * ROWS_PER_TILE, ROWS_PER_TILE)

    @pl.when(c == 0)
    def _():
        pltpu.sync_copy(acc.at[rows], p0.at[rows])

    @pl.when(c == 1)
    def _():
        pltpu.sync_copy(acc.at[rows], p1.at[rows])


def _sc_scatter(yu, yd, yb, eu, edn, eb, tu, tdn, tb, zeros):
    mesh = plsc.VectorSubcoreMesh(core_axis_name="c", subcore_axis_name="s")
    f = pl.kernel(
        _sc_scatter_body,
        out_type=(jax.ShapeDtypeStruct((N_PAD, D), jnp.float32),
                  jax.ShapeDtypeStruct((N_PAD, D), jnp.float32)),
        mesh=mesh,
        scratch_types=[
            pltpu.VMEM_SHARED((N_PAD, D), jnp.float32),
            pltpu.VMEM((KSTG, CH), jnp.int32),
            pltpu.VMEM((KSTG, CH), jnp.int32),
            [pltpu.VMEM((CH, D), jnp.float32)] * NBUF,
            [pltpu.SemaphoreType.DMA] * NBUF,
            [pltpu.SemaphoreType.DMA] * NBUF,
        ],
    )
    return f(yu, yd, yb, eu, edn, eb, tu, tdn, tb, zeros)


def _combine_body(p0_ref, p1_ref, b_ref, o_ref):
    o_ref[...] = p0_ref[...] + p1_ref[...] + b_ref[...]


def _combine(p0, p1, bias):
    blk = 1000
    row_spec = pl.BlockSpec((blk, D), lambda i: (i, 0))
    return pl.pallas_call(
        _combine_body,
        grid=(N // blk,),
        in_specs=[row_spec, row_spec, pl.BlockSpec((1, D), lambda i: (0, 0))],
        out_specs=pl.BlockSpec((blk, D), lambda i: (i, 0)),
        out_shape=jax.ShapeDtypeStruct((N, D), jnp.float32),
    )(p0, p1, bias)


def _split_bulk_tail(idx):
    e = idx.shape[1]
    rows = e // CH
    bulk_rows = (rows // BULK_ALIGN) * BULK_ALIGN
    eb = bulk_rows * CH
    bulk = idx[:, :eb].astype(jnp.int32).reshape(2, bulk_rows, CH)
    tail_e = e - eb
    tail_rows = -(-(tail_e // CH + (1 if tail_e % CH else 0)) // TAIL_ALIGN) * TAIL_ALIGN
    tail_rows = max(tail_rows, TAIL_ALIGN)
    n_pad = tail_rows * CH - tail_e
    pad_s = jnp.asarray(np.arange(n_pad) % (N_PAD - N), jnp.int32)
    pad_d = pad_s + N
    tail = jnp.stack([
        jnp.concatenate([idx[0, eb:].astype(jnp.int32), pad_s]),
        jnp.concatenate([idx[1, eb:].astype(jnp.int32), pad_d]),
    ]).reshape(2, tail_rows, CH)
    return bulk, tail


def kernel(x, up_index, down_index, boundary_index, W_up, W_down, W_b, bias):
    eu, tu = _split_bulk_tail(up_index)
    edn, tdn = _split_bulk_tail(down_index)
    eb, tb = _split_bulk_tail(boundary_index)
    zeros = jnp.zeros((ROWS_PER_TILE, D), jnp.float32)

    yu, yd, yb = _transform(x, W_up, W_down, W_b)
    p0, p1 = _sc_scatter(yu, yd, yb, eu, edn, eb, tu, tdn, tb, zeros)
    return _combine(p0, p1, bias.reshape(1, D))


# speedup vs baseline: 1.0284x; 1.0046x over previous
import numpy as np
import jax
import jax.numpy as jnp
from jax import lax
from jax.experimental import pallas as pl
from jax.experimental.pallas import tpu as pltpu
from jax.experimental.pallas import tpu_sc as plsc

N = 10000
D = 128
NC = 2
NS = 16
NW = NC * NS
CH = 128
NBUF = 2
N_PAD = 10112
ROWS_PER_TILE = N_PAD // NS
KSTG = 40
FRAC0 = 0.5
BULK_ALIGN = 256
TAIL_ALIGN = 256


def _core_split(k_per_pair, align=8):
    a = int(round(FRAC0 * k_per_pair / align)) * align
    a = max(0, min(a, k_per_pair))
    return a, k_per_pair - a


def _matmul_body(x_ref, wu_ref, wd_ref, wb_ref, yu_ref, yd_ref, yb_ref):
    xb = x_ref[...]
    yu_ref[...] = jnp.dot(xb, wu_ref[...], preferred_element_type=jnp.float32)
    yd_ref[...] = jnp.dot(xb, wd_ref[...], preferred_element_type=jnp.float32)
    yb_ref[...] = jnp.dot(xb, wb_ref[...], preferred_element_type=jnp.float32)


def _transform(x, W_up, W_down, W_b):
    blk = 2000
    w_spec = pl.BlockSpec((D, D), lambda i: (0, 0))
    row_spec = pl.BlockSpec((blk, D), lambda i: (i, 0))
    return pl.pallas_call(
        _matmul_body,
        grid=(N // blk,),
        in_specs=[row_spec, w_spec, w_spec, w_spec],
        out_specs=[row_spec, row_spec, row_spec],
        out_shape=[jax.ShapeDtypeStruct((N, D), jnp.float32)] * 3,
    )(x, W_up, W_down, W_b)


def _sc_scatter_body(yu, yd, yb, eu, edn, eb, tu, tdn, tb, zeros,
                     p0, p1, acc, idx_s, idx_d, bufs, gsems, ssems):
    c = lax.axis_index("c")
    s = lax.axis_index("s")

    pltpu.sync_copy(zeros, acc.at[pl.ds(s * ROWS_PER_TILE, ROWS_PER_TILE)])
    plsc.subcore_barrier()

    def run_stage(y, e_hbm, base, k_rows):
        pltpu.sync_copy(e_hbm.at[0, pl.ds(base, k_rows)], idx_s.at[pl.ds(0, k_rows)])
        pltpu.sync_copy(e_hbm.at[1, pl.ds(base, k_rows)], idx_d.at[pl.ds(0, k_rows)])

        def g_start(j, b):
            pltpu.async_copy(y.at[idx_s.at[j]], bufs[b], gsems[b])

        def g_wait(b):
            pltpu.make_async_copy(y.at[idx_s.at[0]], bufs[b], gsems[b]).wait()

        def s_start(j, b):
            pltpu.make_async_copy(bufs[b], acc.at[idx_d.at[j]], ssems[b]).start(add=True)

        def s_wait(b):
            pltpu.make_async_copy(bufs[b], acc.at[idx_d.at[0]], ssems[b]).wait()

        g_start(0, 0)

        def body(i, _):
            for u in range(NBUF):
                j = NBUF * i + u
                b = u
                g_wait(b)
                s_start(j, b)
                bn = 1 - b

                @pl.when(j + 1 < k_rows)
                def _():
                    @pl.when(j >= 1)
                    def _():
                        s_wait(bn)

                    g_start(j + 1, bn)

            return _

        lax.fori_loop(0, k_rows // NBUF, body, None)
        for b in range(NBUF):
            s_wait(b)

    def run_span(y, e_hbm, a_rows, base):
        for st in range(0, a_rows, KSTG):
            k = min(KSTG, a_rows - st)
            run_stage(y, e_hbm, base + st, k)

    work = ((yu, eu, tu), (yd, edn, tdn), (yb, eb, tb))

    @pl.when(c == 0)
    def _():
        for y, e, t in work:
            a, _b = _core_split(e.shape[1] // NS)
            if a:
                run_span(y, e, a, s * a)
            at, _bt = _core_split(t.shape[1] // NS)
            if at:
                run_span(y, t, at, s * at)

    @pl.when(c == 1)
    def _():
        for y, e, t in work:
            a, b = _core_split(e.shape[1] // NS)
            if b:
                run_span(y, e, b, NS * a + s * b)
            at, bt = _core_split(t.shape[1] // NS)
            if bt:
                run_span(y, t, bt, NS * at + s * bt)

    plsc.subcore_barrier()
    rows = pl.ds(s * ROWS_PER_TILE, ROWS_PER_TILE)

    @pl.when(c == 0)
    def _():
        pltpu.sync_copy(acc.at[rows], p0.at[rows])

    @pl.when(c == 1)
    def _():
        pltpu.sync_copy(acc.at[rows], p1.at[rows])


def _sc_scatter(yu, yd, yb, eu, edn, eb, tu, tdn, tb, zeros):
    mesh = plsc.VectorSubcoreMesh(core_axis_name="c", subcore_axis_name="s")
    f = pl.kernel(
        _sc_scatter_body,
        out_type=(jax.ShapeDtypeStruct((N_PAD, D), jnp.float32),
                  jax.ShapeDtypeStruct((N_PAD, D), jnp.float32)),
        mesh=mesh,
        scratch_types=[
            pltpu.VMEM_SHARED((N_PAD, D), jnp.float32),
            pltpu.VMEM((KSTG, CH), jnp.int32),
            pltpu.VMEM((KSTG, CH), jnp.int32),
            [pltpu.VMEM((CH, D), jnp.float32)] * NBUF,
            [pltpu.SemaphoreType.DMA] * NBUF,
            [pltpu.SemaphoreType.DMA] * NBUF,
        ],
    )
    return f(yu, yd, yb, eu, edn, eb, tu, tdn, tb, zeros)


def _combine_body(p0_ref, p1_ref, b_ref, o_ref):
    o_ref[...] = p0_ref[...] + p1_ref[...] + b_ref[...]


def _combine(p0, p1, bias):
    blk = 1000
    row_spec = pl.BlockSpec((blk, D), lambda i: (i, 0))
    return pl.pallas_call(
        _combine_body,
        grid=(N // blk,),
        in_specs=[row_spec, row_spec, pl.BlockSpec((1, D), lambda i: (0, 0))],
        out_specs=pl.BlockSpec((blk, D), lambda i: (i, 0)),
        out_shape=jax.ShapeDtypeStruct((N, D), jnp.float32),
    )(p0, p1, bias)


def _split_bulk_tail(idx):
    e = idx.shape[1]
    rows = e // CH
    bulk_rows = (rows // BULK_ALIGN) * BULK_ALIGN
    eb = bulk_rows * CH
    bulk = idx[:, :eb].astype(jnp.int32).reshape(2, bulk_rows, CH)
    tail_e = e - eb
    tail_rows = -(-(tail_e // CH + (1 if tail_e % CH else 0)) // TAIL_ALIGN) * TAIL_ALIGN
    tail_rows = max(tail_rows, TAIL_ALIGN)
    n_pad = tail_rows * CH - tail_e
    pad_s = jnp.asarray(np.arange(n_pad) % (N_PAD - N), jnp.int32)
    pad_d = pad_s + N
    tail = jnp.stack([
        jnp.concatenate([idx[0, eb:].astype(jnp.int32), pad_s]),
        jnp.concatenate([idx[1, eb:].astype(jnp.int32), pad_d]),
    ]).reshape(2, tail_rows, CH)
    return bulk, tail


def kernel(x, up_index, down_index, boundary_index, W_up, W_down, W_b, bias):
    eu, tu = _split_bulk_tail(up_index)
    edn, tdn = _split_bulk_tail(down_index)
    eb, tb = _split_bulk_tail(boundary_index)
    zeros = jnp.zeros((ROWS_PER_TILE, D), jnp.float32)

    yu, yd, yb = _transform(x, W_up, W_down, W_b)
    p0, p1 = _sc_scatter(yu, yd, yb, eu, edn, eb, tu, tdn, tb, zeros)
    return _combine(p0, p1, bias.reshape(1, D))
